# Initial kernel scaffold; baseline (speedup 1.0000x reference)
import jax
import jax.numpy as jnp
from jax.experimental import pallas as pl


def kernel(x, img, edge_index, Wl1, Wr1, b1, Wl2, Wr2, b2, Wfc, bfc):
    raise NotImplementedError("write your pallas kernel here")


# speedup vs baseline: 4.8693x; 4.8693x over previous
import functools

import jax
import jax.numpy as jnp
from jax import lax
from jax.experimental import pallas as pl
from jax.experimental.pallas import tpu as pltpu
from jax.experimental.pallas import tpu_sc as plsc

N = 10000
E = 320000
NCORE = 2
NSUB = 16
NW = NCORE * NSUB
EPW = E // NW
CHUNK = 80
NCHUNK = EPW // CHUNK
NPAD = 10240
RPT = NPAD // NSUB
BN = 1024

_MESH = dict(core_axis_name="c", subcore_axis_name="s",
             num_cores=NCORE, num_subcores=NSUB)


@functools.lru_cache(maxsize=None)
def _make_segsum():
  mesh = plsc.VectorSubcoreMesh(**_MESH)
  out_type = [jax.ShapeDtypeStruct((NCORE, NPAD, 128), jnp.float32)]
  scratch = [
      pltpu.VMEM_SHARED((NPAD, 128), jnp.float32),
      pltpu.VMEM((CHUNK,), jnp.int32),
      pltpu.VMEM((CHUNK,), jnp.int32),
      pltpu.VMEM((CHUNK, 128), jnp.float32),
      pltpu.SemaphoreType.DMA,
  ]

  def body(p_hbm, src_hbm, dst_hbm, zf_hbm, agg_out, acc_sh, srci_v, dsti_v,
           rows_v, sem):
    c = lax.axis_index("c")
    s = lax.axis_index("s")
    rbase = s * RPT

    pltpu.sync_copy(zf_hbm.at[pl.ds(rbase, RPT)], acc_sh.at[pl.ds(rbase, RPT)])
    plsc.subcore_barrier()

    wbase = (c * NSUB + s) * EPW

    def step(j, carry):
      ebase = wbase + j * CHUNK
      pltpu.sync_copy(src_hbm.at[pl.ds(ebase, CHUNK)], srci_v)
      pltpu.sync_copy(dst_hbm.at[pl.ds(ebase, CHUNK)], dsti_v)
      pltpu.async_copy(p_hbm.at[srci_v], rows_v, sem).wait()
      pltpu.sync_copy(rows_v, acc_sh.at[dsti_v], add=True)
      return carry

    lax.fori_loop(0, NCHUNK, step, 0)
    plsc.subcore_barrier()

    pltpu.sync_copy(acc_sh.at[pl.ds(rbase, RPT)],
                    agg_out.at[c, pl.ds(rbase, RPT)])

  return pl.kernel(body, out_type=out_type, mesh=mesh, scratch_types=scratch)


def _segsum(p, src, dst, zf):
  out = _make_segsum()(p, src, dst, zf)
  return out[0] if isinstance(out, (list, tuple)) else out


@functools.lru_cache(maxsize=None)
def _make_count():
  mesh = plsc.VectorSubcoreMesh(**_MESH)
  out_type = [jax.ShapeDtypeStruct((NCORE, NPAD, 128), jnp.float32)]
  scratch = [
      pltpu.VMEM_SHARED((NPAD, 128), jnp.float32),
      pltpu.VMEM((CHUNK,), jnp.int32),
      pltpu.VMEM((CHUNK, 128), jnp.float32),
  ]

  def body(dst_hbm, ones_hbm, zf_hbm, cnt_out, cnt_sh, dsti_v, ones_v):
    c = lax.axis_index("c")
    s = lax.axis_index("s")
    rbase = s * RPT

    pltpu.sync_copy(zf_hbm.at[pl.ds(rbase, RPT)], cnt_sh.at[pl.ds(rbase, RPT)])
    pltpu.sync_copy(ones_hbm, ones_v)
    plsc.subcore_barrier()

    wbase = (c * NSUB + s) * EPW

    def step(j, carry):
      ebase = wbase + j * CHUNK
      pltpu.sync_copy(dst_hbm.at[pl.ds(ebase, CHUNK)], dsti_v)
      pltpu.sync_copy(ones_v, cnt_sh.at[dsti_v], add=True)
      return carry

    lax.fori_loop(0, NCHUNK, step, 0)
    plsc.subcore_barrier()

    pltpu.sync_copy(cnt_sh.at[pl.ds(rbase, RPT)],
                    cnt_out.at[c, pl.ds(rbase, RPT)])

  return pl.kernel(body, out_type=out_type, mesh=mesh, scratch_types=scratch)


def _count(dst, ones, zf):
  out = _make_count()(dst, ones, zf)
  return out[0] if isinstance(out, (list, tuple)) else out


def _dot(a, b):
  return jnp.dot(a, b, preferred_element_type=jnp.float32)


def _inv_cnt(cntp_ref):
  cnt = cntp_ref[0, :, 0:1] + cntp_ref[1, :, 0:1]
  return 1.0 / jnp.maximum(cnt, 1.0)


def _stage1_body(img_ref, x_ref, wli, wlx, wri, wrx, b_ref, p_ref, r_ref):
  im = img_ref[...]
  xx = x_ref[...]
  p_ref[...] = _dot(im, wli[...]) + _dot(xx, wlx[...])
  r_ref[...] = _dot(im, wri[...]) + _dot(xx, wrx[...]) + b_ref[...]


def _stage2_body(aggp_ref, cntp_ref, r1_ref, wr2, b2_ref, h1_ref, r2_ref):
  agg = aggp_ref[0] + aggp_ref[1]
  h1 = jnp.maximum(agg * _inv_cnt(cntp_ref) + r1_ref[...], 0.0)
  h1_ref[...] = h1
  r2_ref[...] = _dot(h1, wr2[...]) + b2_ref[...]


def _stage3_body(aggp_ref, cntp_ref, r2_ref, wl2, wfc, bfc_ref, o_ref):
  agg = aggp_ref[0] + aggp_ref[1]
  h2 = jnp.maximum(_dot(agg * _inv_cnt(cntp_ref), wl2[...]) + r2_ref[...], 0.0)
  logits = _dot(h2, wfc[...]) + bfc_ref[...]
  m = jnp.max(logits, axis=1, keepdims=True)
  lse = jnp.log(jnp.sum(jnp.exp(logits - m), axis=1, keepdims=True)) + m
  o_ref[...] = logits - lse


def _rows(d):
  return pl.BlockSpec((BN, d), lambda i: (i, 0))


def _part(d):
  return pl.BlockSpec((NCORE, BN, d), lambda i: (0, i, 0))


def _full(r, c):
  return pl.BlockSpec((r, c), lambda i: (0, 0))


_GRID = ((N + BN - 1) // BN,)


def _stage1(img, x, wli, wlx, wri, wrx, b1):
  return pl.pallas_call(
      _stage1_body,
      grid=_GRID,
      in_specs=[_rows(128), _rows(128), _full(128, 128), _full(128, 128),
                _full(128, 128), _full(128, 128), _full(1, 128)],
      out_specs=[_rows(128), _rows(128)],
      out_shape=[jax.ShapeDtypeStruct((N, 128), jnp.float32),
                 jax.ShapeDtypeStruct((N, 128), jnp.float32)],
  )(img, x, wli, wlx, wri, wrx, b1)


def _stage2(aggp, cntp, r1, wr2t, b2):
  return pl.pallas_call(
      _stage2_body,
      grid=_GRID,
      in_specs=[_part(128), _part(128), _rows(128), _full(128, 64),
                _full(1, 64)],
      out_specs=[_rows(128), _rows(64)],
      out_shape=[jax.ShapeDtypeStruct((N, 128), jnp.float32),
                 jax.ShapeDtypeStruct((N, 64), jnp.float32)],
  )(aggp, cntp, r1, wr2t, b2)


def _stage3(aggp, cntp, r2, wl2t, wfct, bfc):
  return pl.pallas_call(
      _stage3_body,
      grid=_GRID,
      in_specs=[_part(128), _part(128), _rows(64), _full(128, 64),
                _full(64, 40), _full(1, 40)],
      out_specs=_rows(40),
      out_shape=jax.ShapeDtypeStruct((N, 40), jnp.float32),
  )(aggp, cntp, r2, wl2t, wfct, bfc)


@jax.jit
def kernel(x, img, edge_index, Wl1, Wr1, b1, Wl2, Wr2, b2, Wfc, bfc):
  src = edge_index[0]
  dst = edge_index[1]
  wl1t = Wl1.T
  wr1t = Wr1.T
  p1, r1 = _stage1(img, x, wl1t[:128], wl1t[128:], wr1t[:128], wr1t[128:],
                   b1.reshape(1, -1))
  zf128 = jnp.zeros((NPAD, 128), jnp.float32)
  ones128 = jnp.ones((CHUNK, 128), jnp.float32)
  cntp = _count(dst, ones128, zf128)
  agg1p = _segsum(p1, src, dst, zf128)
  h1, r2 = _stage2(agg1p, cntp, r1, Wr2.T, b2.reshape(1, -1))
  agg2p = _segsum(h1, src, dst, zf128)
  return _stage3(agg2p, cntp, r2, Wl2.T, Wfc.T, bfc.reshape(1, -1))
